# Initial kernel scaffold; baseline (speedup 1.0000x reference)
import jax
import jax.numpy as jnp
from jax import lax
from jax.experimental import pallas as pl
from jax.experimental.pallas import tpu as pltpu
from jax.experimental.pallas import tpu_sc as plsc

N = 100000
E = 6400000
EPS = 1e-06
LAMB = 1.0
X0 = 0.05

NSUB = 16
NCORE = 2
NTILE = NSUB * NCORE
LANES = 128
R_TOT = E // LANES
KB = 50
CHUNKS = R_TOT // KB
STRIPE = 6272
NPAD = NSUB * STRIPE
S3 = NPAD // NTILE
F32 = jnp.float32


def _step_body(num_in, den_in, edge_hbm, ea_hbm,
               num_out, den_out,
               sig_sh, num_sh, den_sh,
               n0_v, n1_v, d0_v, d1_v, zero_v,
               src_v, dst_v, ea_v, xj_v, xi_v, w_v, wx_v,
               sem_a, sem_b):
    c = lax.axis_index("c")
    s = lax.axis_index("s")
    g = c * NSUB + s
    st = pl.ds(s * STRIPE, STRIPE)

    pltpu.sync_copy(num_in.at[0, st], n0_v)
    pltpu.sync_copy(num_in.at[1, st], n1_v)
    pltpu.sync_copy(den_in.at[0, st], d0_v)
    pltpu.sync_copy(den_in.at[1, st], d1_v)

    def comb(i, carry):
        sl = pl.ds(i * 16, 16)
        numv = n0_v[sl] + n1_v[sl] + X0
        denv = d0_v[sl] + d1_v[sl] + LAMB
        n0_v[sl] = numv / denv
        zero_v[sl] = jnp.zeros((16,), F32)
        return carry

    lax.fori_loop(0, STRIPE // 16, comb, 0)

    pltpu.sync_copy(n0_v, sig_sh.at[st])
    pltpu.sync_copy(zero_v, num_sh.at[st])
    pltpu.sync_copy(zero_v, den_sh.at[st])
    plsc.subcore_barrier()

    n_chunks = (CHUNKS - g + (NTILE - 1)) // NTILE

    def chunk(ch, carry):
        rows = pl.ds((g + ch * NTILE) * KB, KB)
        pltpu.sync_copy(edge_hbm.at[0, rows], src_v)
        pltpu.sync_copy(edge_hbm.at[1, rows], dst_v)
        pltpu.sync_copy(ea_hbm.at[rows], ea_v)
        cp1 = pltpu.async_copy(sig_sh.at[src_v], xj_v, sem_a)
        cp2 = pltpu.async_copy(sig_sh.at[dst_v], xi_v, sem_b)
        cp1.wait()
        cp2.wait()

        def vrow(j, carry2):
            for i in range(LANES // 16):
                sl = pl.ds(i * 16, 16)
                xj = xj_v[j, sl]
                xi = xi_v[j, sl]
                ea = ea_v[j, sl]
                d = xj - xi + EPS
                w = (ea * ea) * (d * d)
                w_v[j, sl] = w
                wx_v[j, sl] = w * xj
            return carry2

        lax.fori_loop(0, KB, vrow, 0)

        pltpu.sync_copy(w_v, den_sh.at[dst_v], add=True)
        pltpu.sync_copy(wx_v, num_sh.at[dst_v], add=True)
        return carry

    lax.fori_loop(0, n_chunks, chunk, 0)

    plsc.subcore_barrier()
    pltpu.sync_copy(num_sh.at[st], num_out.at[c, st])
    pltpu.sync_copy(den_sh.at[st], den_out.at[c, st])


def _fin_body(num_in, den_in, out_hbm, n0_v, n1_v, d0_v, d1_v):
    c = lax.axis_index("c")
    s = lax.axis_index("s")
    g = c * NSUB + s
    st = pl.ds(g * S3, S3)
    pltpu.sync_copy(num_in.at[0, st], n0_v)
    pltpu.sync_copy(num_in.at[1, st], n1_v)
    pltpu.sync_copy(den_in.at[0, st], d0_v)
    pltpu.sync_copy(den_in.at[1, st], d1_v)

    def comb(i, carry):
        sl = pl.ds(i * 16, 16)
        numv = n0_v[sl] + n1_v[sl] + X0
        denv = d0_v[sl] + d1_v[sl] + LAMB
        n0_v[sl] = numv / denv
        return carry

    lax.fori_loop(0, S3 // 16, comb, 0)
    pltpu.sync_copy(n0_v, out_hbm.at[st])


def kernel(signal, edge_attr, edge_index, itr):
    sig = jnp.pad(signal.reshape(N), (0, NPAD - N))
    edge3 = edge_index.reshape(2, R_TOT, LANES)
    ea2 = edge_attr.reshape(R_TOT, LANES)

    num0 = jnp.stack([sig - X0, jnp.zeros_like(sig)])
    den0 = jnp.zeros((2, NPAD), F32)

    mesh = plsc.VectorSubcoreMesh(core_axis_name="c", subcore_axis_name="s")
    part = jax.ShapeDtypeStruct((2, NPAD), F32)
    step = pl.kernel(
        _step_body,
        out_type=(part, part),
        mesh=mesh,
        scratch_types=[
            pltpu.VMEM_SHARED((NPAD,), F32),
            pltpu.VMEM_SHARED((NPAD,), F32),
            pltpu.VMEM_SHARED((NPAD,), F32),
            pltpu.VMEM((STRIPE,), F32),
            pltpu.VMEM((STRIPE,), F32),
            pltpu.VMEM((STRIPE,), F32),
            pltpu.VMEM((STRIPE,), F32),
            pltpu.VMEM((STRIPE,), F32),
            pltpu.VMEM((KB, LANES), jnp.int32),
            pltpu.VMEM((KB, LANES), jnp.int32),
            pltpu.VMEM((KB, LANES), F32),
            pltpu.VMEM((KB, LANES), F32),
            pltpu.VMEM((KB, LANES), F32),
            pltpu.VMEM((KB, LANES), F32),
            pltpu.VMEM((KB, LANES), F32),
            pltpu.SemaphoreType.DMA,
            pltpu.SemaphoreType.DMA,
        ],
    )

    def body(_, carry):
        num_p, den_p = carry
        return step(num_p, den_p, edge3, ea2)

    num_f, den_f = lax.fori_loop(0, itr, body, (num0, den0))

    fin = pl.kernel(
        _fin_body,
        out_type=jax.ShapeDtypeStruct((NPAD,), F32),
        mesh=mesh,
        scratch_types=[
            pltpu.VMEM((S3,), F32),
            pltpu.VMEM((S3,), F32),
            pltpu.VMEM((S3,), F32),
            pltpu.VMEM((S3,), F32),
        ],
    )
    sig_out = fin(num_f, den_f)
    return sig_out[:N].reshape(N, 1)


# speedup vs baseline: 331.8413x; 331.8413x over previous
import jax
import jax.numpy as jnp
from jax import lax
from jax.experimental import pallas as pl
from jax.experimental.pallas import tpu as pltpu
from jax.experimental.pallas import tpu_sc as plsc

N = 100000
E = 6400000
EPS = 1e-06
LAMB = 1.0
X0 = 0.05

NSUB = 16
NCORE = 2
NTILE = NSUB * NCORE
C = 6400
CHUNKS = E // C
STRIPE = 6272
NPAD = NSUB * STRIPE
S3 = NPAD // NTILE
F32 = jnp.float32


def _step_body(num_in, den_in, edge_hbm, ea_hbm,
               num_out, den_out,
               sig_sh, num_sh, den_sh,
               n0_v, n1_v, d0_v, d1_v, zero_v,
               src_v, dst_v, ea_v, xj_v, xi_v, w_v, wx_v,
               sem_a, sem_b):
    c = lax.axis_index("c")
    s = lax.axis_index("s")
    g = c * NSUB + s
    st = pl.ds(s * STRIPE, STRIPE)

    st2 = pl.ds(NPAD + s * STRIPE, STRIPE)
    pltpu.sync_copy(num_in.at[st], n0_v)
    pltpu.sync_copy(num_in.at[st2], n1_v)
    pltpu.sync_copy(den_in.at[st], d0_v)
    pltpu.sync_copy(den_in.at[st2], d1_v)

    def comb(i, carry):
        sl = pl.ds(i * 16, 16)
        numv = n0_v[sl] + n1_v[sl] + X0
        denv = d0_v[sl] + d1_v[sl] + LAMB
        n0_v[sl] = numv / denv
        zero_v[sl] = jnp.zeros((16,), F32)
        return carry

    lax.fori_loop(0, STRIPE // 16, comb, 0)

    pltpu.sync_copy(n0_v, sig_sh.at[st])
    pltpu.sync_copy(zero_v, num_sh.at[st])
    pltpu.sync_copy(zero_v, den_sh.at[st])
    plsc.subcore_barrier()

    n_chunks = (CHUNKS - g + (NTILE - 1)) // NTILE

    def chunk(ch, carry):
        base = (g + ch * NTILE) * C
        pltpu.sync_copy(edge_hbm.at[pl.ds(base, C)], src_v)
        pltpu.sync_copy(edge_hbm.at[pl.ds(E + base, C)], dst_v)
        pltpu.sync_copy(ea_hbm.at[pl.ds(base, C)], ea_v)
        cp1 = pltpu.async_copy(sig_sh.at[src_v], xj_v, sem_a)
        cp2 = pltpu.async_copy(sig_sh.at[dst_v], xi_v, sem_b)
        cp1.wait()
        cp2.wait()

        def vrow(j, carry2):
            sl = pl.ds(j * 16, 16)
            xj = xj_v[sl]
            xi = xi_v[sl]
            ea = ea_v[sl]
            d = xj - xi + EPS
            w = (ea * ea) * (d * d)
            w_v[sl] = w
            wx_v[sl] = w * xj
            return carry2

        lax.fori_loop(0, C // 16, vrow, 0)

        pltpu.sync_copy(w_v, den_sh.at[dst_v], add=True)
        pltpu.sync_copy(wx_v, num_sh.at[dst_v], add=True)
        return carry

    lax.fori_loop(0, n_chunks, chunk, 0)

    plsc.subcore_barrier()
    sto = pl.ds(c * NPAD + s * STRIPE, STRIPE)
    pltpu.sync_copy(num_sh.at[st], num_out.at[sto])
    pltpu.sync_copy(den_sh.at[st], den_out.at[sto])


def _fin_body(num_in, den_in, out_hbm, n0_v, n1_v, d0_v, d1_v):
    c = lax.axis_index("c")
    s = lax.axis_index("s")

    @pl.when(c == 0)
    def _():
        st = pl.ds(s * STRIPE, STRIPE)
        st2 = pl.ds(NPAD + s * STRIPE, STRIPE)
        pltpu.sync_copy(num_in.at[st], n0_v)
        pltpu.sync_copy(num_in.at[st2], n1_v)
        pltpu.sync_copy(den_in.at[st], d0_v)
        pltpu.sync_copy(den_in.at[st2], d1_v)

        def comb(i, carry):
            sl = pl.ds(i * 16, 16)
            numv = n0_v[sl] + n1_v[sl] + X0
            denv = d0_v[sl] + d1_v[sl] + LAMB
            n0_v[sl] = numv / denv
            return carry

        lax.fori_loop(0, STRIPE // 16, comb, 0)
        pltpu.sync_copy(n0_v, out_hbm.at[st])


def kernel(signal, edge_attr, edge_index, itr):
    sig = jnp.pad(signal.reshape(N), (0, NPAD - N))
    edge1 = edge_index.reshape(2 * E)
    ea1 = edge_attr.reshape(E)

    num0 = jnp.concatenate([sig - X0, jnp.zeros_like(sig)])
    den0 = jnp.zeros((2 * NPAD,), F32)

    mesh = plsc.VectorSubcoreMesh(core_axis_name="c", subcore_axis_name="s")
    part = jax.ShapeDtypeStruct((2 * NPAD,), F32)
    step = pl.kernel(
        _step_body,
        out_type=(part, part),
        mesh=mesh,
        scratch_types=[
            pltpu.VMEM_SHARED((NPAD,), F32),
            pltpu.VMEM_SHARED((NPAD,), F32),
            pltpu.VMEM_SHARED((NPAD,), F32),
            pltpu.VMEM((STRIPE,), F32),
            pltpu.VMEM((STRIPE,), F32),
            pltpu.VMEM((STRIPE,), F32),
            pltpu.VMEM((STRIPE,), F32),
            pltpu.VMEM((STRIPE,), F32),
            pltpu.VMEM((C,), jnp.int32),
            pltpu.VMEM((C,), jnp.int32),
            pltpu.VMEM((C,), F32),
            pltpu.VMEM((C,), F32),
            pltpu.VMEM((C,), F32),
            pltpu.VMEM((C,), F32),
            pltpu.VMEM((C,), F32),
            pltpu.SemaphoreType.DMA,
            pltpu.SemaphoreType.DMA,
        ],
    )

    def body(_, carry):
        num_p, den_p = carry
        return step(num_p, den_p, edge1, ea1)

    num_f, den_f = lax.fori_loop(0, itr, body, (num0, den0))

    fin = pl.kernel(
        _fin_body,
        out_type=jax.ShapeDtypeStruct((NPAD,), F32),
        mesh=mesh,
        scratch_types=[
            pltpu.VMEM((STRIPE,), F32),
            pltpu.VMEM((STRIPE,), F32),
            pltpu.VMEM((STRIPE,), F32),
            pltpu.VMEM((STRIPE,), F32),
        ],
    )
    sig_out = fin(num_f, den_f)
    return sig_out[:N].reshape(N, 1)


# speedup vs baseline: 353.6249x; 1.0656x over previous
import jax
import jax.numpy as jnp
from jax import lax
from jax.experimental import pallas as pl
from jax.experimental.pallas import tpu as pltpu
from jax.experimental.pallas import tpu_sc as plsc

N = 100000
E = 6400000
EPS = 1e-06
LAMB = 1.0
X0 = 0.05

NSUB = 16
NCORE = 2
NTILE = NSUB * NCORE
C = 2000
CHUNKS = E // C
NCH = CHUNKS // NTILE
STRIPE = 6272
NPAD = NSUB * STRIPE
S3 = NPAD // NTILE
F32 = jnp.float32


def _step_body(num_in, den_in, edge_hbm, ea_hbm,
               num_out, den_out,
               sig_sh, num_sh, den_sh,
               n0_v, n1_v, d0_v, d1_v, zero_v,
               src0, src1, src2, dst0, dst1, dst2, eav0, eav1, eav2,
               xj0, xj1, xj2, xi0, xi1, xi2,
               w0, w1, w2, wx0, wx1, wx2,
               sl0, sl1, sl2, sg0, sg1, sg2, ss0, ss1, ss2):
    src_v = (src0, src1, src2)
    dst_v = (dst0, dst1, dst2)
    ea_v = (eav0, eav1, eav2)
    xj_v = (xj0, xj1, xj2)
    xi_v = (xi0, xi1, xi2)
    w_v = (w0, w1, w2)
    wx_v = (wx0, wx1, wx2)
    sem_l = (sl0, sl1, sl2)
    sem_g = (sg0, sg1, sg2)
    sem_s = (ss0, ss1, ss2)
    c = lax.axis_index("c")
    s = lax.axis_index("s")
    g = c * NSUB + s
    st = pl.ds(s * STRIPE, STRIPE)

    st2 = pl.ds(NPAD + s * STRIPE, STRIPE)
    pltpu.sync_copy(num_in.at[st], n0_v)
    pltpu.sync_copy(num_in.at[st2], n1_v)
    pltpu.sync_copy(den_in.at[st], d0_v)
    pltpu.sync_copy(den_in.at[st2], d1_v)

    def comb(i, carry):
        sl = pl.ds(i * 16, 16)
        numv = n0_v[sl] + n1_v[sl] + X0
        denv = d0_v[sl] + d1_v[sl] + LAMB
        n0_v[sl] = numv / denv
        zero_v[sl] = jnp.zeros((16,), F32)
        return carry

    lax.fori_loop(0, STRIPE // 16, comb, 0)

    pltpu.sync_copy(n0_v, sig_sh.at[st])
    pltpu.sync_copy(zero_v, num_sh.at[st])
    pltpu.sync_copy(zero_v, den_sh.at[st])
    plsc.subcore_barrier()

    def issue_load(t, b):
        base = (g + t * NTILE) * C
        pltpu.async_copy(edge_hbm.at[pl.ds(base, C)], src_v[b], sem_l[b])
        pltpu.async_copy(edge_hbm.at[pl.ds(E + base, C)], dst_v[b], sem_l[b])
        pltpu.async_copy(ea_hbm.at[pl.ds(base, C)], ea_v[b], sem_l[b])

    def wait_load(t, b):
        base = (g + t * NTILE) * C
        pltpu.make_async_copy(edge_hbm.at[pl.ds(base, C)], src_v[b], sem_l[b]).wait()
        pltpu.make_async_copy(edge_hbm.at[pl.ds(E + base, C)], dst_v[b], sem_l[b]).wait()
        pltpu.make_async_copy(ea_hbm.at[pl.ds(base, C)], ea_v[b], sem_l[b]).wait()

    def issue_gather(b):
        pltpu.async_copy(sig_sh.at[src_v[b]], xj_v[b], sem_g[b])
        pltpu.async_copy(sig_sh.at[dst_v[b]], xi_v[b], sem_g[b])

    def wait_gather(b):
        pltpu.make_async_copy(sig_sh.at[src_v[b]], xj_v[b], sem_g[b]).wait()
        pltpu.make_async_copy(sig_sh.at[dst_v[b]], xi_v[b], sem_g[b]).wait()

    def issue_scatter(b):
        pltpu.async_copy(w_v[b], den_sh.at[dst_v[b]], sem_s[b], add=True)
        pltpu.async_copy(wx_v[b], num_sh.at[dst_v[b]], sem_s[b], add=True)

    def wait_scatter(b):
        pltpu.make_async_copy(w_v[b], den_sh.at[dst_v[b]], sem_s[b]).wait()
        pltpu.make_async_copy(wx_v[b], num_sh.at[dst_v[b]], sem_s[b]).wait()

    def compute(b):
        xjr, xir, ear, wr, wxr = xj_v[b], xi_v[b], ea_v[b], w_v[b], wx_v[b]

        def vstep(j, carry2):
            for k in range(4):
                sl = pl.ds(j * 64 + k * 16, 16)
                xj = xjr[sl]
                xi = xir[sl]
                ea = ear[sl]
                d = xj - xi + EPS
                w = (ea * ea) * (d * d)
                wr[sl] = w
                wxr[sl] = w * xj
            return carry2

        lax.fori_loop(0, C // 64, vstep, 0)

    issue_load(0, 0)
    issue_load(1, 1)
    wait_load(0, 0)
    issue_gather(0)

    def pipe(i, carry):
        for b in range(3):
            t = i * 3 + b
            b1 = (b + 1) % 3
            b2 = (b + 2) % 3

            @pl.when(t <= NCH - 3)
            def _():
                issue_load(t + 2, b2)

            @pl.when(t <= NCH - 2)
            def _():
                wait_load(t + 1, b1)
                issue_gather(b1)

            @pl.when(t <= NCH - 1)
            def _():
                wait_gather(b)
                compute(b)
                pltpu.sync_copy(w_v[b], den_sh.at[dst_v[b]], add=True)
                pltpu.sync_copy(wx_v[b], num_sh.at[dst_v[b]], add=True)
        return carry

    lax.fori_loop(0, (NCH + 3) // 3, pipe, 0)

    plsc.subcore_barrier()
    sto = pl.ds(c * NPAD + s * STRIPE, STRIPE)
    pltpu.sync_copy(num_sh.at[st], num_out.at[sto])
    pltpu.sync_copy(den_sh.at[st], den_out.at[sto])


def _fin_body(num_in, den_in, out_hbm, n0_v, n1_v, d0_v, d1_v):
    c = lax.axis_index("c")
    s = lax.axis_index("s")

    @pl.when(c == 0)
    def _():
        st = pl.ds(s * STRIPE, STRIPE)
        st2 = pl.ds(NPAD + s * STRIPE, STRIPE)
        pltpu.sync_copy(num_in.at[st], n0_v)
        pltpu.sync_copy(num_in.at[st2], n1_v)
        pltpu.sync_copy(den_in.at[st], d0_v)
        pltpu.sync_copy(den_in.at[st2], d1_v)

        def comb(i, carry):
            sl = pl.ds(i * 16, 16)
            numv = n0_v[sl] + n1_v[sl] + X0
            denv = d0_v[sl] + d1_v[sl] + LAMB
            n0_v[sl] = numv / denv
            return carry

        lax.fori_loop(0, STRIPE // 16, comb, 0)
        pltpu.sync_copy(n0_v, out_hbm.at[st])


def kernel(signal, edge_attr, edge_index, itr):
    sig = jnp.pad(signal.reshape(N), (0, NPAD - N))
    edge1 = edge_index.reshape(2 * E)
    ea1 = edge_attr.reshape(E)

    num0 = jnp.concatenate([sig - X0, jnp.zeros_like(sig)])
    den0 = jnp.zeros((2 * NPAD,), F32)

    mesh = plsc.VectorSubcoreMesh(core_axis_name="c", subcore_axis_name="s")
    part = jax.ShapeDtypeStruct((2 * NPAD,), F32)
    step = pl.kernel(
        _step_body,
        out_type=(part, part),
        mesh=mesh,
        scratch_types=[
            pltpu.VMEM_SHARED((NPAD,), F32),
            pltpu.VMEM_SHARED((NPAD,), F32),
            pltpu.VMEM_SHARED((NPAD,), F32),
            pltpu.VMEM((STRIPE,), F32),
            pltpu.VMEM((STRIPE,), F32),
            pltpu.VMEM((STRIPE,), F32),
            pltpu.VMEM((STRIPE,), F32),
            pltpu.VMEM((STRIPE,), F32),
        ] + [pltpu.VMEM((C,), jnp.int32) for _ in range(6)]
          + [pltpu.VMEM((C,), F32) for _ in range(15)]
          + [pltpu.SemaphoreType.DMA for _ in range(9)],
    )

    def body(_, carry):
        num_p, den_p = carry
        return step(num_p, den_p, edge1, ea1)

    num_f, den_f = lax.fori_loop(0, itr, body, (num0, den0))

    fin = pl.kernel(
        _fin_body,
        out_type=jax.ShapeDtypeStruct((NPAD,), F32),
        mesh=mesh,
        scratch_types=[
            pltpu.VMEM((STRIPE,), F32),
            pltpu.VMEM((STRIPE,), F32),
            pltpu.VMEM((STRIPE,), F32),
            pltpu.VMEM((STRIPE,), F32),
        ],
    )
    sig_out = fin(num_f, den_f)
    return sig_out[:N].reshape(N, 1)


# speedup vs baseline: 363.0349x; 1.0266x over previous
import jax
import jax.numpy as jnp
from jax import lax
from jax.experimental import pallas as pl
from jax.experimental.pallas import tpu as pltpu
from jax.experimental.pallas import tpu_sc as plsc

N = 100000
E = 6400000
EPS = 1e-06
LAMB = 1.0
X0 = 0.05

NSUB = 16
NCORE = 2
NTILE = NSUB * NCORE
C = 2000
CHUNKS = E // C
NCH = CHUNKS // NTILE
STRIPE = 6272
NPAD = NSUB * STRIPE
S3 = NPAD // NTILE
F32 = jnp.float32


def _step_body(num_in, den_in, edge_hbm, ea_hbm,
               num_out, den_out,
               sig_sh, num_sh, den_sh,
               n0_v, n1_v, d0_v, d1_v, zero_v,
               src0, src1, src2, dst0, dst1, dst2, eav0, eav1, eav2,
               xj0, xj1, xj2, xi0, xi1, xi2,
               w0, w1, w2, wx0, wx1, wx2,
               sl0, sl1, sl2, sg0, sg1, sg2, ss0, ss1, ss2):
    src_v = (src0, src1, src2)
    dst_v = (dst0, dst1, dst2)
    ea_v = (eav0, eav1, eav2)
    xj_v = (xj0, xj1, xj2)
    xi_v = (xi0, xi1, xi2)
    w_v = (w0, w1, w2)
    wx_v = (wx0, wx1, wx2)
    sem_l = (sl0, sl1, sl2)
    sem_g = (sg0, sg1, sg2)
    sem_s = (ss0, ss1, ss2)
    c = lax.axis_index("c")
    s = lax.axis_index("s")
    g = c * NSUB + s
    st = pl.ds(s * STRIPE, STRIPE)

    st2 = pl.ds(NPAD + s * STRIPE, STRIPE)
    pltpu.sync_copy(num_in.at[st], n0_v)
    pltpu.sync_copy(num_in.at[st2], n1_v)
    pltpu.sync_copy(den_in.at[st], d0_v)
    pltpu.sync_copy(den_in.at[st2], d1_v)

    def comb(i, carry):
        sl = pl.ds(i * 16, 16)
        numv = n0_v[sl] + n1_v[sl] + X0
        denv = d0_v[sl] + d1_v[sl] + LAMB
        n0_v[sl] = numv / denv
        zero_v[sl] = jnp.zeros((16,), F32)
        return carry

    lax.fori_loop(0, STRIPE // 16, comb, 0)

    pltpu.sync_copy(n0_v, sig_sh.at[st])
    pltpu.sync_copy(zero_v, num_sh.at[st])
    pltpu.sync_copy(zero_v, den_sh.at[st])
    plsc.subcore_barrier()

    def issue_load(t, b):
        base = (g + t * NTILE) * C
        pltpu.async_copy(edge_hbm.at[pl.ds(base, C)], src_v[b], sem_l[b])
        pltpu.async_copy(edge_hbm.at[pl.ds(E + base, C)], dst_v[b], sem_l[b])
        pltpu.async_copy(ea_hbm.at[pl.ds(base, C)], ea_v[b], sem_l[b])

    def wait_load(t, b):
        base = (g + t * NTILE) * C
        pltpu.make_async_copy(edge_hbm.at[pl.ds(base, C)], src_v[b], sem_l[b]).wait()
        pltpu.make_async_copy(edge_hbm.at[pl.ds(E + base, C)], dst_v[b], sem_l[b]).wait()
        pltpu.make_async_copy(ea_hbm.at[pl.ds(base, C)], ea_v[b], sem_l[b]).wait()

    def issue_gather(b):
        pltpu.async_copy(sig_sh.at[src_v[b]], xj_v[b], sem_g[b])
        pltpu.async_copy(sig_sh.at[dst_v[b]], xi_v[b], sem_g[b])

    def wait_gather(b):
        pltpu.make_async_copy(sig_sh.at[src_v[b]], xj_v[b], sem_g[b]).wait()
        pltpu.make_async_copy(sig_sh.at[dst_v[b]], xi_v[b], sem_g[b]).wait()

    def issue_scatter(b):
        pltpu.async_copy(w_v[b], den_sh.at[dst_v[b]], sem_s[b], add=True)
        pltpu.async_copy(wx_v[b], num_sh.at[dst_v[b]], sem_s[b], add=True)

    def wait_scatter(b):
        pltpu.make_async_copy(w_v[b], den_sh.at[dst_v[b]], sem_s[b]).wait()
        pltpu.make_async_copy(wx_v[b], num_sh.at[dst_v[b]], sem_s[b]).wait()

    def compute(b):
        xjr, xir, ear, wr, wxr = xj_v[b], xi_v[b], ea_v[b], w_v[b], wx_v[b]

        def vstep(j, carry2):
            for k in range(4):
                sl = pl.ds(j * 64 + k * 16, 16)
                xj = xjr[sl]
                xi = xir[sl]
                ea = ear[sl]
                d = xj - xi + EPS
                w = (ea * ea) * (d * d)
                wr[sl] = w
                wxr[sl] = w * xj
            return carry2

        lax.fori_loop(0, C // 64, vstep, 0)

    issue_load(0, 0)
    issue_load(1, 1)
    wait_load(0, 0)
    issue_gather(0)

    def pipe(i, carry):
        for b in range(3):
            t = i * 3 + b
            b1 = (b + 1) % 3
            b2 = (b + 2) % 3

            @pl.when(t <= NCH - 3)
            def _():
                issue_load(t + 2, b2)

            @pl.when(t <= NCH - 2)
            def _():
                wait_load(t + 1, b1)
                issue_gather(b1)

            @pl.when(t <= NCH - 1)
            def _():
                wait_gather(b)
                compute(b)
                cp_d = pltpu.async_copy(w_v[b], den_sh.at[dst_v[b]],
                                        sem_s[b], add=True)
                cp_n = pltpu.async_copy(wx_v[b], num_sh.at[dst_v[b]],
                                        sem_s[b], add=True)
                cp_d.wait()
                cp_n.wait()
        return carry

    lax.fori_loop(0, (NCH + 3) // 3, pipe, 0)

    plsc.subcore_barrier()
    sto = pl.ds(c * NPAD + s * STRIPE, STRIPE)
    pltpu.sync_copy(num_sh.at[st], num_out.at[sto])
    pltpu.sync_copy(den_sh.at[st], den_out.at[sto])


def _fin_body(num_in, den_in, out_hbm, n0_v, n1_v, d0_v, d1_v):
    c = lax.axis_index("c")
    s = lax.axis_index("s")

    @pl.when(c == 0)
    def _():
        st = pl.ds(s * STRIPE, STRIPE)
        st2 = pl.ds(NPAD + s * STRIPE, STRIPE)
        pltpu.sync_copy(num_in.at[st], n0_v)
        pltpu.sync_copy(num_in.at[st2], n1_v)
        pltpu.sync_copy(den_in.at[st], d0_v)
        pltpu.sync_copy(den_in.at[st2], d1_v)

        def comb(i, carry):
            sl = pl.ds(i * 16, 16)
            numv = n0_v[sl] + n1_v[sl] + X0
            denv = d0_v[sl] + d1_v[sl] + LAMB
            n0_v[sl] = numv / denv
            return carry

        lax.fori_loop(0, STRIPE // 16, comb, 0)
        pltpu.sync_copy(n0_v, out_hbm.at[st])


def kernel(signal, edge_attr, edge_index, itr):
    sig = jnp.pad(signal.reshape(N), (0, NPAD - N))
    edge1 = edge_index.reshape(2 * E)
    ea1 = edge_attr.reshape(E)

    num0 = jnp.concatenate([sig - X0, jnp.zeros_like(sig)])
    den0 = jnp.zeros((2 * NPAD,), F32)

    mesh = plsc.VectorSubcoreMesh(core_axis_name="c", subcore_axis_name="s")
    part = jax.ShapeDtypeStruct((2 * NPAD,), F32)
    step = pl.kernel(
        _step_body,
        out_type=(part, part),
        mesh=mesh,
        scratch_types=[
            pltpu.VMEM_SHARED((NPAD,), F32),
            pltpu.VMEM_SHARED((NPAD,), F32),
            pltpu.VMEM_SHARED((NPAD,), F32),
            pltpu.VMEM((STRIPE,), F32),
            pltpu.VMEM((STRIPE,), F32),
            pltpu.VMEM((STRIPE,), F32),
            pltpu.VMEM((STRIPE,), F32),
            pltpu.VMEM((STRIPE,), F32),
        ] + [pltpu.VMEM((C,), jnp.int32) for _ in range(6)]
          + [pltpu.VMEM((C,), F32) for _ in range(15)]
          + [pltpu.SemaphoreType.DMA for _ in range(9)],
    )

    def body(_, carry):
        num_p, den_p = carry
        return step(num_p, den_p, edge1, ea1)

    num_f, den_f = lax.fori_loop(0, itr, body, (num0, den0))

    fin = pl.kernel(
        _fin_body,
        out_type=jax.ShapeDtypeStruct((NPAD,), F32),
        mesh=mesh,
        scratch_types=[
            pltpu.VMEM((STRIPE,), F32),
            pltpu.VMEM((STRIPE,), F32),
            pltpu.VMEM((STRIPE,), F32),
            pltpu.VMEM((STRIPE,), F32),
        ],
    )
    sig_out = fin(num_f, den_f)
    return sig_out[:N].reshape(N, 1)
